# baseline (speedup 1.0000x reference)
import jax
import jax.numpy as jnp
from jax.experimental import pallas as pl
from jax.experimental.pallas import tpu as pltpu
from jax.experimental.pallas import tpu_sc as plsc

B, L, D = 1024, 50, 128
LP = 64
NEG_SLOPE = 0.2
MASK_VAL = -9e15
SPS = 4
GATHER_W = 128


def _sc_gather(emb, idx_flat):
    n = idx_flat.shape[1]
    mesh = plsc.VectorSubcoreMesh(core_axis_name="core",
                                  subcore_axis_name="subcore")

    @pl.kernel(out_type=jax.ShapeDtypeStruct((n, D), emb.dtype), mesh=mesh)
    def gather_kernel(emb_hbm, i_hbm, o_hbm):
        def body(i_vmem, o_vmem):
            pltpu.sync_copy(emb_hbm.at[i_vmem.at[0]], o_vmem)

        pltpu.emit_pipeline(
            body,
            grid=(n // GATHER_W,),
            in_specs=[pl.BlockSpec((1, GATHER_W), lambda i: (0, i))],
            out_specs=[pl.BlockSpec((GATHER_W, D), lambda i: (i, 0))],
            core_axis_name=("core", "subcore"),
            dimension_semantics=(pltpu.PARALLEL,),
        )(i_hbm, o_hbm)

    return gather_kernel(emb, idx_flat)


def _attn_body(a4_ref, h_ref, adj_ref, out_ref, adq_ref):
    a4 = a4_ref[...].astype(jnp.bfloat16)
    rowhalf = jax.lax.broadcasted_iota(jnp.int32, (2 * LP, 2 * LP), 0) // LP
    colid = jax.lax.broadcasted_iota(jnp.int32, (2 * LP, 2 * LP), 1)
    own = (rowhalf == colid // LP) & (colid % LP < L)
    own_f = own.astype(jnp.float32)

    adq_ref[...] = jnp.zeros((SPS, 2 * LP, 2 * LP), jnp.int32)
    for s in range(SPS):
        adq_ref[s, 0:L, 0:L] = adj_ref[2 * s]
        adq_ref[s, LP:LP + L, LP:LP + L] = adj_ref[2 * s + 1]

    for s in range(SPS):
        hp = h_ref[s].astype(jnp.bfloat16)
        adjq = adq_ref[s]
        acc = jnp.full((2 * LP, 2 * LP), MASK_VAL, dtype=jnp.float32)
        for k in range(4):
            g = hp * a4[k:k + 1, :]
            e = jax.lax.dot_general(
                hp, g, (((1,), (1,)), ((), ())),
                preferred_element_type=jnp.float32)
            e = jnp.where(e >= 0, e, NEG_SLOPE * e)
            acc = jnp.where(adjq == k + 1, e, acc)
        m = jnp.max(acc, axis=1, keepdims=True)
        p = jnp.where(m == MASK_VAL, own_f, jnp.exp(acc - m))
        ssum = jnp.sum(p, axis=1, keepdims=True)
        w = (p / ssum).astype(jnp.bfloat16)
        o = jax.lax.dot_general(
            w, hp, (((1,), (0,)), ((), ())),
            preferred_element_type=jnp.float32)
        out_ref[2 * s] = o[0:L]
        out_ref[2 * s + 1] = o[LP:LP + L]


def kernel(total_items, total_adj, embedding, a_0, a_1, a_2, a_3):
    idx = jnp.pad(total_items.astype(jnp.int32), ((0, 0), (0, LP - L)))
    h = _sc_gather(embedding, idx.reshape(1, B * LP))
    h = h.reshape(B // 2, 2 * LP, D)
    a4 = jnp.concatenate([a_0.T, a_1.T, a_2.T, a_3.T], axis=0)
    return pl.pallas_call(
        _attn_body,
        grid=(B // (2 * SPS),),
        in_specs=[
            pl.BlockSpec((4, D), lambda i: (0, 0)),
            pl.BlockSpec((SPS, 2 * LP, D), lambda i: (i, 0, 0)),
            pl.BlockSpec((2 * SPS, L, L), lambda i: (i, 0, 0)),
        ],
        out_specs=pl.BlockSpec((2 * SPS, L, D), lambda i: (i, 0, 0)),
        out_shape=jax.ShapeDtypeStruct((B, L, D), jnp.float32),
        scratch_shapes=[pltpu.VMEM((SPS, 2 * LP, 2 * LP), jnp.int32)],
    )(a4, h, total_adj)


# speedup vs baseline: 2.8886x; 2.8886x over previous
import jax
import jax.numpy as jnp
from jax.experimental import pallas as pl
from jax.experimental.pallas import tpu as pltpu
from jax.experimental.pallas import tpu_sc as plsc

B, L, D = 1024, 50, 128
LP = 64
NEG_SLOPE = 0.2
MASK_VAL = -9e15
SPS = 4
GATHER_W = 128


def _sc_gather(emb, idx_flat):
    n = idx_flat.shape[1]
    mesh = plsc.VectorSubcoreMesh(core_axis_name="core",
                                  subcore_axis_name="subcore")

    @pl.kernel(out_type=jax.ShapeDtypeStruct((n, D), emb.dtype), mesh=mesh)
    def gather_kernel(emb_hbm, i_hbm, o_hbm):
        def body(i_vmem, o_vmem):
            pltpu.sync_copy(emb_hbm.at[i_vmem.at[0]], o_vmem)

        pltpu.emit_pipeline(
            body,
            grid=(n // GATHER_W,),
            in_specs=[pl.BlockSpec((1, GATHER_W), lambda i: (0, i))],
            out_specs=[pl.BlockSpec((GATHER_W, D), lambda i: (i, 0))],
            core_axis_name=("core", "subcore"),
            dimension_semantics=(pltpu.PARALLEL,),
        )(i_hbm, o_hbm)

    return gather_kernel(emb, idx_flat)


def _attn_body(a4_ref, h_ref, adj_ref, out_ref, adq_ref):
    a4 = a4_ref[...].astype(jnp.bfloat16)
    rowhalf = jax.lax.broadcasted_iota(jnp.int32, (2 * LP, 2 * LP), 0) // LP
    colid = jax.lax.broadcasted_iota(jnp.int32, (2 * LP, 2 * LP), 1)
    own = (rowhalf == colid // LP) & (colid % LP < L)
    own_f = own.astype(jnp.float32)

    adq_ref[...] = jnp.zeros((SPS, 2 * LP, 2 * LP), jnp.int32)
    for s in range(SPS):
        adq_ref[s, 0:L, 0:L] = adj_ref[2 * s]
        adq_ref[s, LP:LP + L, LP:LP + L] = adj_ref[2 * s + 1]

    for s in range(SPS):
        hp = h_ref[s].astype(jnp.bfloat16)
        adjq = adq_ref[s]
        acc = jnp.full((2 * LP, 2 * LP), MASK_VAL, dtype=jnp.float32)
        for k in range(4):
            g = hp * a4[k:k + 1, :]
            e = jax.lax.dot_general(
                hp, g, (((1,), (1,)), ((), ())),
                preferred_element_type=jnp.float32)
            e = jnp.where(e >= 0, e, NEG_SLOPE * e)
            acc = jnp.where(adjq == k + 1, e, acc)
        m = jnp.max(acc, axis=1, keepdims=True)
        p = jnp.where(m == MASK_VAL, own_f, jnp.exp(acc - m))
        ssum = jnp.sum(p, axis=1, keepdims=True)
        w = (p / ssum).astype(jnp.bfloat16)
        o = jax.lax.dot_general(
            w, hp, (((1,), (0,)), ((), ())),
            preferred_element_type=jnp.float32)
        out_ref[2 * s] = o[0:L]
        out_ref[2 * s + 1] = o[LP:LP + L]


def kernel(total_items, total_adj, embedding, a_0, a_1, a_2, a_3):
    items32 = total_items.astype(jnp.int32)
    idx = jnp.concatenate([items32, items32[:, :LP - L]], axis=1)
    h = _sc_gather(embedding, idx.reshape(1, B * LP))
    h = h.reshape(B // 2, 2 * LP, D)
    a4 = jnp.concatenate([a_0.T, a_1.T, a_2.T, a_3.T], axis=0)
    return pl.pallas_call(
        _attn_body,
        grid=(B // (2 * SPS),),
        in_specs=[
            pl.BlockSpec((4, D), lambda i: (0, 0)),
            pl.BlockSpec((SPS, 2 * LP, D), lambda i: (i, 0, 0)),
            pl.BlockSpec((2 * SPS, L, L), lambda i: (i, 0, 0)),
        ],
        out_specs=pl.BlockSpec((2 * SPS, L, D), lambda i: (i, 0, 0)),
        out_shape=jax.ShapeDtypeStruct((B, L, D), jnp.float32),
        scratch_shapes=[pltpu.VMEM((SPS, 2 * LP, 2 * LP), jnp.int32)],
    )(a4, h, total_adj)


# speedup vs baseline: 3.9778x; 1.3770x over previous
import jax
import jax.numpy as jnp
from jax.experimental import pallas as pl
from jax.experimental.pallas import tpu as pltpu
from jax.experimental.pallas import tpu_sc as plsc

B, L, D = 1024, 50, 128
LP = 64
NEG_SLOPE = 0.2
MASK_VAL = -9e15
SPS = 4
GATHER_W = 128


def _sc_gather(emb, idx_flat):
    n = idx_flat.shape[1]
    mesh = plsc.VectorSubcoreMesh(core_axis_name="core",
                                  subcore_axis_name="subcore")

    @pl.kernel(out_type=jax.ShapeDtypeStruct((n, D), emb.dtype), mesh=mesh)
    def gather_kernel(emb_hbm, i_hbm, o_hbm):
        def body(i_vmem, o_vmem):
            pltpu.sync_copy(emb_hbm.at[i_vmem.at[0]], o_vmem)

        pltpu.emit_pipeline(
            body,
            grid=(n // GATHER_W,),
            in_specs=[pl.BlockSpec((1, GATHER_W), lambda i: (0, i))],
            out_specs=[pl.BlockSpec((GATHER_W, D), lambda i: (i, 0))],
            core_axis_name=("core", "subcore"),
            dimension_semantics=(pltpu.PARALLEL,),
        )(i_hbm, o_hbm)

    return gather_kernel(emb, idx_flat)


def _attn_body(a4_ref, h_ref, adj_ref, out_ref, adq_ref, e_ref, w_ref):
    a4 = a4_ref[...].astype(jnp.bfloat16)
    rowhalf = jax.lax.broadcasted_iota(jnp.int32, (2 * LP, 2 * LP), 0) // LP
    colid = jax.lax.broadcasted_iota(jnp.int32, (2 * LP, 2 * LP), 1)
    own = (rowhalf == colid // LP) & (colid % LP < L)
    own_f = own.astype(jnp.float32)

    adq_ref[...] = jnp.zeros((SPS, 2 * LP, 2 * LP), jnp.int32)
    for s in range(SPS):
        adq_ref[s, 0:L, 0:L] = adj_ref[2 * s]
        adq_ref[s, LP:LP + L, LP:LP + L] = adj_ref[2 * s + 1]

    for s in range(SPS):
        hp = h_ref[s].astype(jnp.bfloat16)
        for k in range(4):
            g = hp * a4[k:k + 1, :]
            e_ref[s, k] = jax.lax.dot_general(
                hp, g, (((1,), (1,)), ((), ())),
                preferred_element_type=jnp.float32)

    e = e_ref[...]
    e = jnp.where(e >= 0, e, NEG_SLOPE * e)
    adjq = adq_ref[...]
    acc = jnp.full((SPS, 2 * LP, 2 * LP), MASK_VAL, dtype=jnp.float32)
    for k in range(4):
        acc = jnp.where(adjq == k + 1, e[:, k], acc)
    m = jnp.max(acc, axis=2, keepdims=True)
    p = jnp.where(m == MASK_VAL, own_f[None], jnp.exp(acc - m))
    ssum = jnp.sum(p, axis=2, keepdims=True)
    w_ref[...] = (p / ssum).astype(jnp.bfloat16)

    for s in range(SPS):
        o = jax.lax.dot_general(
            w_ref[s], h_ref[s].astype(jnp.bfloat16), (((1,), (0,)), ((), ())),
            preferred_element_type=jnp.float32)
        out_ref[2 * s] = o[0:L]
        out_ref[2 * s + 1] = o[LP:LP + L]


def kernel(total_items, total_adj, embedding, a_0, a_1, a_2, a_3):
    items32 = total_items.astype(jnp.int32)
    idx = jnp.concatenate([items32, items32[:, :LP - L]], axis=1)
    h = _sc_gather(embedding, idx.reshape(1, B * LP))
    h = h.reshape(B // 2, 2 * LP, D)
    a4 = jnp.concatenate([a_0.T, a_1.T, a_2.T, a_3.T], axis=0)
    return pl.pallas_call(
        _attn_body,
        grid=(B // (2 * SPS),),
        in_specs=[
            pl.BlockSpec((4, D), lambda i: (0, 0)),
            pl.BlockSpec((SPS, 2 * LP, D), lambda i: (i, 0, 0)),
            pl.BlockSpec((2 * SPS, L, L), lambda i: (i, 0, 0)),
        ],
        out_specs=pl.BlockSpec((2 * SPS, L, D), lambda i: (i, 0, 0)),
        out_shape=jax.ShapeDtypeStruct((B, L, D), jnp.float32),
        scratch_shapes=[
            pltpu.VMEM((SPS, 2 * LP, 2 * LP), jnp.int32),
            pltpu.VMEM((SPS, 4, 2 * LP, 2 * LP), jnp.float32),
            pltpu.VMEM((SPS, 2 * LP, 2 * LP), jnp.bfloat16),
        ],
    )(a4, h, total_adj)
